# baseline (speedup 1.0000x reference)
import functools

import jax
import jax.numpy as jnp
from jax import lax
from jax.experimental import pallas as pl
from jax.experimental.pallas import tpu as pltpu
from jax.experimental.pallas import tpu_sc as plsc

N = 50000
E = 1600000
R = 46
B = 30
H = 16
C = 2

NUM_CORES = 2
NUM_SUBCORES = 16
NW = NUM_CORES * NUM_SUBCORES
EP = E // NW
CH = 2000
NCH = EP // CH
_SC_PARAMS = pltpu.CompilerParams(use_tc_tiling_on_sc=False)

NHP = 802816
NROWS = NHP // H
NPAD = 51200
ROWS_PER_TILE = NPAD // NUM_SUBCORES
BLKC = 2048


_BW = 57344
_NJ = NHP // _BW


def _w1_body(att_ref, basis_ref, *out_refs):
    res = jnp.dot(att_ref[...], basis_ref[...],
                  preferred_element_type=jnp.float32)
    for t in range(R):
        out_refs[t][...] = res[t:t + 1, :].reshape(-1)


def _build_w1(att1, basis_flat):
    return pl.pallas_call(
        _w1_body,
        grid=(_NJ,),
        in_specs=[
            pl.BlockSpec((R, B), lambda j: (0, 0)),
            pl.BlockSpec((B, _BW), lambda j: (0, j)),
        ],
        out_specs=[pl.BlockSpec((_BW,), lambda j: (j,))] * R,
        out_shape=[jax.ShapeDtypeStruct((NHP,), jnp.float32)] * R,
    )(att1, basis_flat)


def _idx_body(s_ref, t_ref, r1_ref, r2_ref):
    s = s_ref[...]
    t = t_ref[...]
    r1_ref[...] = t * NROWS + s
    r2_ref[...] = t * NPAD + s


def _build_indices(src2d, typ2d):
    rows, cols = src2d.shape
    blk = 128
    grid = rows // blk
    return pl.pallas_call(
        _idx_body,
        grid=(grid,),
        in_specs=[pl.BlockSpec((blk, cols), lambda i: (i, 0))] * 2,
        out_specs=[pl.BlockSpec((blk, cols), lambda i: (i, 0))] * 2,
        out_shape=[jax.ShapeDtypeStruct((rows, cols), jnp.int32)] * 2,
    )(src2d, typ2d)


REM = ROWS_PER_TILE - CH


def _edge_agg1(ridx_hbm, dst_hbm, w_hbm, zrows_hbm, zcnt_hbm,
               sums_out, cnt_out,
               ridx_v, dst_v, rows_v, ridx2_v, dst2_v, rows2_v, ones_v,
               acc_sh, cnt_sh, sem, sem2, isem, isem2):
    c = lax.axis_index("c")
    s = lax.axis_index("s")
    wid = c * NUM_SUBCORES + s

    r0 = s * ROWS_PER_TILE
    pltpu.sync_copy(zrows_hbm, rows_v)
    pltpu.sync_copy(rows_v, acc_sh.at[pl.ds(r0, CH)])
    pltpu.sync_copy(rows_v.at[pl.ds(0, REM)], acc_sh.at[pl.ds(r0 + CH, REM)])
    pltpu.sync_copy(zcnt_hbm, ones_v)
    pltpu.sync_copy(ones_v, cnt_sh.at[pl.ds(r0, CH)])
    pltpu.sync_copy(ones_v.at[pl.ds(0, REM)], cnt_sh.at[pl.ds(r0 + CH, REM)])

    for i in range(CH // 16):
        ones_v[pl.ds(i * 16, 16)] = jnp.ones((16,), jnp.float32)
    plsc.subcore_barrier()

    base0 = wid * EP
    bufs = ((ridx_v, dst_v, rows_v, sem),
            (ridx2_v, dst2_v, rows2_v, sem2))
    pltpu.sync_copy(ridx_hbm.at[pl.ds(base0, CH)], ridx_v)
    pltpu.sync_copy(dst_hbm.at[pl.ds(base0, CH)], dst_v)
    gd = [pltpu.async_copy(w_hbm.at[ridx_v], rows_v, sem), None]
    for k in range(NCH):
        p = k & 1
        ridx_p, dst_p, rows_p, _ = bufs[p]
        ridx_n, dst_n, rows_n, sem_n = bufs[1 - p]
        ia = ib = None
        if k + 1 < NCH:
            base_n = base0 + (k + 1) * CH
            ia = pltpu.async_copy(ridx_hbm.at[pl.ds(base_n, CH)], ridx_n,
                                  isem)
            ib = pltpu.async_copy(dst_hbm.at[pl.ds(base_n, CH)], dst_n,
                                  isem2)
        gd[p].wait()
        if k + 1 < NCH:
            ia.wait()
            ib.wait()
            gd[1 - p] = pltpu.async_copy(w_hbm.at[ridx_n], rows_n, sem_n)
        pltpu.sync_copy(rows_p, acc_sh.at[dst_p], add=True)
        pltpu.sync_copy(ones_v, cnt_sh.at[dst_p], add=True)

    plsc.subcore_barrier()
    pltpu.sync_copy(acc_sh.at[pl.ds(r0, CH)], rows_v)
    pltpu.sync_copy(rows_v, sums_out.at[c, pl.ds(r0, CH)])
    pltpu.sync_copy(acc_sh.at[pl.ds(r0 + CH, REM)], rows_v.at[pl.ds(0, REM)])
    pltpu.sync_copy(rows_v.at[pl.ds(0, REM)],
                    sums_out.at[c, pl.ds(r0 + CH, REM)])
    pltpu.sync_copy(cnt_sh.at[pl.ds(r0, CH)], ones_v)
    pltpu.sync_copy(ones_v, cnt_out.at[pl.ds(c * NPAD + r0, CH)])
    pltpu.sync_copy(cnt_sh.at[pl.ds(r0 + CH, REM)], ones_v.at[pl.ds(0, REM)])
    pltpu.sync_copy(ones_v.at[pl.ds(0, REM)],
                    cnt_out.at[pl.ds(c * NPAD + r0 + CH, REM)])


def _run_edge_agg1(ridx1, dst, w_rows):
    zrows = jnp.zeros((CH, H), jnp.float32)
    zcnt = jnp.zeros((CH,), jnp.float32)
    mesh = plsc.VectorSubcoreMesh(core_axis_name="c", subcore_axis_name="s")
    f = functools.partial(
        pl.kernel,
        out_type=[
            jax.ShapeDtypeStruct((NUM_CORES, NPAD, H), jnp.float32),
            jax.ShapeDtypeStruct((NUM_CORES * NPAD,), jnp.float32),
        ],
        mesh=mesh,
        scratch_types=[
            pltpu.VMEM((CH,), jnp.int32),
            pltpu.VMEM((CH,), jnp.int32),
            pltpu.VMEM((CH, H), jnp.float32),
            pltpu.VMEM((CH,), jnp.int32),
            pltpu.VMEM((CH,), jnp.int32),
            pltpu.VMEM((CH, H), jnp.float32),
            pltpu.VMEM((CH,), jnp.float32),
            pltpu.VMEM_SHARED((NPAD, H), jnp.float32),
            pltpu.VMEM_SHARED((NPAD,), jnp.float32),
            pltpu.SemaphoreType.DMA,
            pltpu.SemaphoreType.DMA,
            pltpu.SemaphoreType.DMA,
            pltpu.SemaphoreType.DMA,
        ],
        compiler_params=_SC_PARAMS,
    )(_edge_agg1)
    return f(ridx1, dst, w_rows, zrows, zcnt)


def _edge_agg2(ridx_hbm, dst_hbm, t2a_hbm, t2b_hbm, zcnt_hbm,
               acca_out, accb_out,
               ridx_v, dst_v, msga_v, msgb_v, ridx2_v, dst2_v, msga2_v,
               msgb2_v, acca_sh, accb_sh, sem, semb, sem2, semb2, isem,
               isem2):
    c = lax.axis_index("c")
    s = lax.axis_index("s")
    wid = c * NUM_SUBCORES + s

    r0 = s * ROWS_PER_TILE
    pltpu.sync_copy(zcnt_hbm, msga_v)
    pltpu.sync_copy(msga_v, acca_sh.at[pl.ds(r0, CH)])
    pltpu.sync_copy(msga_v.at[pl.ds(0, REM)], acca_sh.at[pl.ds(r0 + CH, REM)])
    pltpu.sync_copy(msga_v, accb_sh.at[pl.ds(r0, CH)])
    pltpu.sync_copy(msga_v.at[pl.ds(0, REM)], accb_sh.at[pl.ds(r0 + CH, REM)])
    plsc.subcore_barrier()

    base0 = wid * EP
    bufs = ((ridx_v, dst_v, msga_v, msgb_v, sem, semb),
            (ridx2_v, dst2_v, msga2_v, msgb2_v, sem2, semb2))
    pltpu.sync_copy(ridx_hbm.at[pl.ds(base0, CH)], ridx_v)
    pltpu.sync_copy(dst_hbm.at[pl.ds(base0, CH)], dst_v)
    gd = [(pltpu.async_copy(t2a_hbm.at[ridx_v], msga_v, sem),
           pltpu.async_copy(t2b_hbm.at[ridx_v], msgb_v, semb)), None]
    for k in range(NCH):
        p = k & 1
        _, dst_p, msga_p, msgb_p, _, _ = bufs[p]
        ridx_n, dst_n, msga_n, msgb_n, sem_n, semb_n = bufs[1 - p]
        ia = ib = None
        if k + 1 < NCH:
            base_n = base0 + (k + 1) * CH
            ia = pltpu.async_copy(ridx_hbm.at[pl.ds(base_n, CH)], ridx_n,
                                  isem)
            ib = pltpu.async_copy(dst_hbm.at[pl.ds(base_n, CH)], dst_n,
                                  isem2)
        gd[p][0].wait()
        gd[p][1].wait()
        if k + 1 < NCH:
            ia.wait()
            ib.wait()
            gd[1 - p] = (pltpu.async_copy(t2a_hbm.at[ridx_n], msga_n, sem_n),
                         pltpu.async_copy(t2b_hbm.at[ridx_n], msgb_n, semb_n))
        pltpu.sync_copy(msga_p, acca_sh.at[dst_p], add=True)
        pltpu.sync_copy(msgb_p, accb_sh.at[dst_p], add=True)

    plsc.subcore_barrier()
    pltpu.sync_copy(acca_sh.at[pl.ds(r0, CH)], msga_v)
    pltpu.sync_copy(msga_v, acca_out.at[pl.ds(c * NPAD + r0, CH)])
    pltpu.sync_copy(acca_sh.at[pl.ds(r0 + CH, REM)], msga_v.at[pl.ds(0, REM)])
    pltpu.sync_copy(msga_v.at[pl.ds(0, REM)],
                    acca_out.at[pl.ds(c * NPAD + r0 + CH, REM)])
    pltpu.sync_copy(accb_sh.at[pl.ds(r0, CH)], msgb_v)
    pltpu.sync_copy(msgb_v, accb_out.at[pl.ds(c * NPAD + r0, CH)])
    pltpu.sync_copy(accb_sh.at[pl.ds(r0 + CH, REM)], msgb_v.at[pl.ds(0, REM)])
    pltpu.sync_copy(msgb_v.at[pl.ds(0, REM)],
                    accb_out.at[pl.ds(c * NPAD + r0 + CH, REM)])


def _run_edge_agg2(ridx2, dst, t2a, t2b):
    zcnt = jnp.zeros((CH,), jnp.float32)
    mesh = plsc.VectorSubcoreMesh(core_axis_name="c", subcore_axis_name="s")
    f = functools.partial(
        pl.kernel,
        out_type=[
            jax.ShapeDtypeStruct((NUM_CORES * NPAD,), jnp.float32),
            jax.ShapeDtypeStruct((NUM_CORES * NPAD,), jnp.float32),
        ],
        mesh=mesh,
        scratch_types=[
            pltpu.VMEM((CH,), jnp.int32),
            pltpu.VMEM((CH,), jnp.int32),
            pltpu.VMEM((CH,), jnp.float32),
            pltpu.VMEM((CH,), jnp.float32),
            pltpu.VMEM((CH,), jnp.int32),
            pltpu.VMEM((CH,), jnp.int32),
            pltpu.VMEM((CH,), jnp.float32),
            pltpu.VMEM((CH,), jnp.float32),
            pltpu.VMEM_SHARED((NPAD,), jnp.float32),
            pltpu.VMEM_SHARED((NPAD,), jnp.float32),
            pltpu.SemaphoreType.DMA,
            pltpu.SemaphoreType.DMA,
            pltpu.SemaphoreType.DMA,
            pltpu.SemaphoreType.DMA,
            pltpu.SemaphoreType.DMA,
            pltpu.SemaphoreType.DMA,
        ],
        compiler_params=_SC_PARAMS,
    )(_edge_agg2)
    return f(ridx2, dst, t2a, t2b, zcnt)


def _fin1_body(sums_ref, cnt_ref, rootT_ref, biasT_ref, a2x_ref, b2y_ref,
               xt_ref, w2t_ref):
    sums = sums_ref[0] + sums_ref[1]
    cinv = 1.0 / jnp.maximum(cnt_ref[0] + cnt_ref[1], 1.0)
    xt = jax.nn.relu(sums.T * cinv + rootT_ref[...] + biasT_ref[...])
    xt_ref[...] = xt
    w2t_ref[...] = jnp.dot(a2x_ref[...], b2y_ref[...],
                           preferred_element_type=jnp.float32)


def _finalize1(sums_p, cnt_p, root1t, bias1t, att2x, basis2y):
    grid = NPAD // BLKC
    return pl.pallas_call(
        _fin1_body,
        grid=(grid,),
        in_specs=[
            pl.BlockSpec((NUM_CORES, BLKC, H), lambda i: (0, i, 0)),
            pl.BlockSpec((NUM_CORES, 1, BLKC), lambda i: (0, 0, i)),
            pl.BlockSpec((H, BLKC), lambda i: (0, i)),
            pl.BlockSpec((H, 1), lambda i: (0, 0)),
            pl.BlockSpec((R * C, B * C), lambda i: (0, 0)),
            pl.BlockSpec((B * C, H), lambda i: (0, 0)),
        ],
        out_specs=[
            pl.BlockSpec((H, BLKC), lambda i: (0, i)),
            pl.BlockSpec((R * C, H), lambda i: (0, 0)),
        ],
        out_shape=[
            jax.ShapeDtypeStruct((H, NPAD), jnp.float32),
            jax.ShapeDtypeStruct((R * C, H), jnp.float32),
        ],
    )(sums_p, cnt_p.reshape(NUM_CORES, 1, NPAD), root1t, bias1t, att2x,
      basis2y)


_BLK2 = NPAD // 2


def _t2_body(xt_ref, w2t3_ref, a_ref, b_ref):
    xt = xt_ref[...]
    a_ref[...] = jnp.dot(w2t3_ref[0, 0:1, :], xt,
                         preferred_element_type=jnp.float32).reshape(-1)
    b_ref[...] = jnp.dot(w2t3_ref[0, 1:2, :], xt,
                         preferred_element_type=jnp.float32).reshape(-1)


def _build_t2(xt, w2t):
    return pl.pallas_call(
        _t2_body,
        grid=(2, R),
        in_specs=[
            pl.BlockSpec((H, _BLK2), lambda i, t: (0, i)),
            pl.BlockSpec((1, C, H), lambda i, t: (t, 0, 0)),
        ],
        out_specs=[
            pl.BlockSpec((_BLK2,), lambda i, t: (t * 2 + i,)),
            pl.BlockSpec((_BLK2,), lambda i, t: (t * 2 + i,)),
        ],
        out_shape=[
            jax.ShapeDtypeStruct((R * NPAD,), jnp.float32),
            jax.ShapeDtypeStruct((R * NPAD,), jnp.float32),
        ],
    )(xt, w2t.reshape(R, C, H))


def _fin2_body(acca_ref, accb_ref, cnt_ref, xt_ref, root2t_ref, bias2t_ref,
               out_ref):
    acc = jnp.concatenate(
        [acca_ref[0] + acca_ref[1], accb_ref[0] + accb_ref[1]], axis=0)
    cinv = 1.0 / jnp.maximum(cnt_ref[0] + cnt_ref[1], 1.0)
    z = acc * cinv + jnp.dot(root2t_ref[...], xt_ref[...],
                             preferred_element_type=jnp.float32)         + bias2t_ref[...]
    m = jnp.max(z, axis=0, keepdims=True)
    lse = m + jnp.log(jnp.sum(jnp.exp(z - m), axis=0, keepdims=True))
    out_ref[...] = z - lse


def _finalize2(acc2a_p, acc2b_p, cnt_p, xt, root2t, bias2t):
    grid = NPAD // BLKC
    return pl.pallas_call(
        _fin2_body,
        grid=(grid,),
        in_specs=[
            pl.BlockSpec((NUM_CORES, 1, BLKC), lambda i: (0, 0, i)),
            pl.BlockSpec((NUM_CORES, 1, BLKC), lambda i: (0, 0, i)),
            pl.BlockSpec((NUM_CORES, 1, BLKC), lambda i: (0, 0, i)),
            pl.BlockSpec((H, BLKC), lambda i: (0, i)),
            pl.BlockSpec((C, H), lambda i: (0, 0)),
            pl.BlockSpec((C, 1), lambda i: (0, 0)),
        ],
        out_specs=pl.BlockSpec((C, BLKC), lambda i: (0, i)),
        out_shape=jax.ShapeDtypeStruct((C, NPAD), jnp.float32),
    )(acc2a_p.reshape(NUM_CORES, 1, NPAD), acc2b_p.reshape(NUM_CORES, 1, NPAD),
      cnt_p.reshape(NUM_CORES, 1, NPAD), xt, root2t, bias2t)


def kernel(edge_index, edge_type, edge_norm, basis1, att1, root1, bias1,
           basis2, att2, root2, bias2):
    del edge_norm
    src = edge_index[0]
    dst = edge_index[1]

    w1segs = _build_w1(att1, basis1.reshape(B, N * H))
    w_rows = jnp.concatenate(w1segs).reshape(R * NROWS, H)

    src2d = src.reshape(12800, 125)
    typ2d = edge_type.reshape(12800, 125)
    ridx1_2d, ridx2_2d = _build_indices(src2d, typ2d)
    ridx1 = ridx1_2d.reshape(E)
    ridx2 = ridx2_2d.reshape(E)

    sums_p, cnt_p = _run_edge_agg1(ridx1, dst, w_rows)
    cnt_p = cnt_p.reshape(NUM_CORES, NPAD)

    att2x = (att2[:, None, :, None] *
             jnp.eye(C, dtype=jnp.float32)[None, :, None, :]).reshape(
                 R * C, B * C)
    basis2y = jnp.transpose(basis2, (0, 2, 1)).reshape(B * C, H)
    root1t = jnp.pad(root1.T, ((0, 0), (0, NPAD - N)))

    xt, w2t = _finalize1(sums_p, cnt_p, root1t, bias1.reshape(H, 1),
                         att2x, basis2y)
    t2a, t2b = _build_t2(xt, w2t)

    acc2a_p, acc2b_p = _run_edge_agg2(ridx2, dst, t2a, t2b)

    outt = _finalize2(acc2a_p, acc2b_p, cnt_p, xt, root2.T,
                      bias2.reshape(C, 1))
    return outt.T[:N]


# speedup vs baseline: 2.2687x; 2.2687x over previous
import functools

import jax
import jax.numpy as jnp
from jax import lax
from jax.experimental import pallas as pl
from jax.experimental.pallas import tpu as pltpu
from jax.experimental.pallas import tpu_sc as plsc

N = 50000
E = 1600000
R = 46
B = 30
H = 16
C = 2

NUM_CORES = 2
NUM_SUBCORES = 16
NW = NUM_CORES * NUM_SUBCORES
EP = E // NW
CH = 2000
NCH = EP // CH
_SC_PARAMS = pltpu.CompilerParams(use_tc_tiling_on_sc=False)

NHP = 802816
NROWS = NHP // H
NPAD = 51200
ROWS_PER_TILE = NPAD // NUM_SUBCORES
BLKC = 2048


def _w1_body(att_ref, basis_ref, out_ref):
    res = jnp.dot(att_ref[0], basis_ref[...],
                  preferred_element_type=jnp.float32)
    out_ref[...] = res.reshape(-1)


_BW = 200704
_NJ = NHP // _BW


def _build_w1(att3, basis_pad):
    return pl.pallas_call(
        _w1_body,
        grid=(_NJ, R),
        in_specs=[
            pl.BlockSpec((1, 1, B), lambda j, t: (t, 0, 0)),
            pl.BlockSpec((B, _BW), lambda j, t: (0, j)),
        ],
        out_specs=pl.BlockSpec((_BW,), lambda j, t: (t * _NJ + j,)),
        out_shape=jax.ShapeDtypeStruct((R * NHP,), jnp.float32),
        compiler_params=pltpu.CompilerParams(vmem_limit_bytes=120 * 1024 * 1024),
    )(att3, basis_pad)


def _idx_body(s_ref, t_ref, r1_ref, r2_ref):
    s = s_ref[...]
    t = t_ref[...]
    r1_ref[...] = t * NROWS + s
    r2_ref[...] = t * NPAD + s


def _build_indices(src2d, typ2d):
    rows, cols = src2d.shape
    blk = 128
    grid = rows // blk
    return pl.pallas_call(
        _idx_body,
        grid=(grid,),
        in_specs=[pl.BlockSpec((blk, cols), lambda i: (i, 0))] * 2,
        out_specs=[pl.BlockSpec((blk, cols), lambda i: (i, 0))] * 2,
        out_shape=[jax.ShapeDtypeStruct((rows, cols), jnp.int32)] * 2,
    )(src2d, typ2d)


REM = ROWS_PER_TILE - CH


def _edge_agg1(ridx_hbm, dst_hbm, w_hbm, zrows_hbm, zcnt_hbm,
               sums_out, cnt_out,
               ridx_v, dst_v, rows_v, ridx2_v, dst2_v, rows2_v, ones_v,
               acc_sh, cnt_sh, sem, sem2, isem, isem2):
    c = lax.axis_index("c")
    s = lax.axis_index("s")
    wid = c * NUM_SUBCORES + s

    r0 = s * ROWS_PER_TILE
    pltpu.sync_copy(zrows_hbm, rows_v)
    pltpu.sync_copy(rows_v, acc_sh.at[pl.ds(r0, CH)])
    pltpu.sync_copy(rows_v.at[pl.ds(0, REM)], acc_sh.at[pl.ds(r0 + CH, REM)])
    pltpu.sync_copy(zcnt_hbm, ones_v)
    pltpu.sync_copy(ones_v, cnt_sh.at[pl.ds(r0, CH)])
    pltpu.sync_copy(ones_v.at[pl.ds(0, REM)], cnt_sh.at[pl.ds(r0 + CH, REM)])

    for i in range(CH // 16):
        ones_v[pl.ds(i * 16, 16)] = jnp.ones((16,), jnp.float32)
    plsc.subcore_barrier()

    base0 = wid * EP
    bufs = ((ridx_v, dst_v, rows_v, sem),
            (ridx2_v, dst2_v, rows2_v, sem2))
    pltpu.sync_copy(ridx_hbm.at[pl.ds(base0, CH)], ridx_v)
    pltpu.sync_copy(dst_hbm.at[pl.ds(base0, CH)], dst_v)
    gd = [pltpu.async_copy(w_hbm.at[ridx_v], rows_v, sem), None]
    for k in range(NCH):
        p = k & 1
        ridx_p, dst_p, rows_p, _ = bufs[p]
        ridx_n, dst_n, rows_n, sem_n = bufs[1 - p]
        ia = ib = None
        if k + 1 < NCH:
            base_n = base0 + (k + 1) * CH
            ia = pltpu.async_copy(ridx_hbm.at[pl.ds(base_n, CH)], ridx_n,
                                  isem)
            ib = pltpu.async_copy(dst_hbm.at[pl.ds(base_n, CH)], dst_n,
                                  isem2)
        gd[p].wait()
        if k + 1 < NCH:
            ia.wait()
            ib.wait()
            gd[1 - p] = pltpu.async_copy(w_hbm.at[ridx_n], rows_n, sem_n)
        pltpu.sync_copy(rows_p, acc_sh.at[dst_p], add=True)
        pltpu.sync_copy(ones_v, cnt_sh.at[dst_p], add=True)

    plsc.subcore_barrier()
    pltpu.sync_copy(acc_sh.at[pl.ds(r0, CH)], rows_v)
    pltpu.sync_copy(rows_v, sums_out.at[c, pl.ds(r0, CH)])
    pltpu.sync_copy(acc_sh.at[pl.ds(r0 + CH, REM)], rows_v.at[pl.ds(0, REM)])
    pltpu.sync_copy(rows_v.at[pl.ds(0, REM)],
                    sums_out.at[c, pl.ds(r0 + CH, REM)])
    pltpu.sync_copy(cnt_sh.at[pl.ds(r0, CH)], ones_v)
    pltpu.sync_copy(ones_v, cnt_out.at[pl.ds(c * NPAD + r0, CH)])
    pltpu.sync_copy(cnt_sh.at[pl.ds(r0 + CH, REM)], ones_v.at[pl.ds(0, REM)])
    pltpu.sync_copy(ones_v.at[pl.ds(0, REM)],
                    cnt_out.at[pl.ds(c * NPAD + r0 + CH, REM)])


def _run_edge_agg1(ridx1, dst, w_rows):
    zrows = jnp.zeros((CH, H), jnp.float32)
    zcnt = jnp.zeros((CH,), jnp.float32)
    mesh = plsc.VectorSubcoreMesh(core_axis_name="c", subcore_axis_name="s")
    f = functools.partial(
        pl.kernel,
        out_type=[
            jax.ShapeDtypeStruct((NUM_CORES, NPAD, H), jnp.float32),
            jax.ShapeDtypeStruct((NUM_CORES * NPAD,), jnp.float32),
        ],
        mesh=mesh,
        scratch_types=[
            pltpu.VMEM((CH,), jnp.int32),
            pltpu.VMEM((CH,), jnp.int32),
            pltpu.VMEM((CH, H), jnp.float32),
            pltpu.VMEM((CH,), jnp.int32),
            pltpu.VMEM((CH,), jnp.int32),
            pltpu.VMEM((CH, H), jnp.float32),
            pltpu.VMEM((CH,), jnp.float32),
            pltpu.VMEM_SHARED((NPAD, H), jnp.float32),
            pltpu.VMEM_SHARED((NPAD,), jnp.float32),
            pltpu.SemaphoreType.DMA,
            pltpu.SemaphoreType.DMA,
            pltpu.SemaphoreType.DMA,
            pltpu.SemaphoreType.DMA,
        ],
        compiler_params=_SC_PARAMS,
    )(_edge_agg1)
    return f(ridx1, dst, w_rows, zrows, zcnt)


def _edge_agg2(ridx_hbm, dst_hbm, t2a_hbm, t2b_hbm, zcnt_hbm,
               acca_out, accb_out,
               ridx_v, dst_v, msga_v, msgb_v, ridx2_v, dst2_v, msga2_v,
               msgb2_v, acca_sh, accb_sh, sem, semb, sem2, semb2, isem,
               isem2):
    c = lax.axis_index("c")
    s = lax.axis_index("s")
    wid = c * NUM_SUBCORES + s

    r0 = s * ROWS_PER_TILE
    pltpu.sync_copy(zcnt_hbm, msga_v)
    pltpu.sync_copy(msga_v, acca_sh.at[pl.ds(r0, CH)])
    pltpu.sync_copy(msga_v.at[pl.ds(0, REM)], acca_sh.at[pl.ds(r0 + CH, REM)])
    pltpu.sync_copy(msga_v, accb_sh.at[pl.ds(r0, CH)])
    pltpu.sync_copy(msga_v.at[pl.ds(0, REM)], accb_sh.at[pl.ds(r0 + CH, REM)])
    plsc.subcore_barrier()

    base0 = wid * EP
    bufs = ((ridx_v, dst_v, msga_v, msgb_v, sem, semb),
            (ridx2_v, dst2_v, msga2_v, msgb2_v, sem2, semb2))
    pltpu.sync_copy(ridx_hbm.at[pl.ds(base0, CH)], ridx_v)
    pltpu.sync_copy(dst_hbm.at[pl.ds(base0, CH)], dst_v)
    gd = [(pltpu.async_copy(t2a_hbm.at[ridx_v], msga_v, sem),
           pltpu.async_copy(t2b_hbm.at[ridx_v], msgb_v, semb)), None]
    for k in range(NCH):
        p = k & 1
        _, dst_p, msga_p, msgb_p, _, _ = bufs[p]
        ridx_n, dst_n, msga_n, msgb_n, sem_n, semb_n = bufs[1 - p]
        ia = ib = None
        if k + 1 < NCH:
            base_n = base0 + (k + 1) * CH
            ia = pltpu.async_copy(ridx_hbm.at[pl.ds(base_n, CH)], ridx_n,
                                  isem)
            ib = pltpu.async_copy(dst_hbm.at[pl.ds(base_n, CH)], dst_n,
                                  isem2)
        gd[p][0].wait()
        gd[p][1].wait()
        if k + 1 < NCH:
            ia.wait()
            ib.wait()
            gd[1 - p] = (pltpu.async_copy(t2a_hbm.at[ridx_n], msga_n, sem_n),
                         pltpu.async_copy(t2b_hbm.at[ridx_n], msgb_n, semb_n))
        pltpu.sync_copy(msga_p, acca_sh.at[dst_p], add=True)
        pltpu.sync_copy(msgb_p, accb_sh.at[dst_p], add=True)

    plsc.subcore_barrier()
    pltpu.sync_copy(acca_sh.at[pl.ds(r0, CH)], msga_v)
    pltpu.sync_copy(msga_v, acca_out.at[pl.ds(c * NPAD + r0, CH)])
    pltpu.sync_copy(acca_sh.at[pl.ds(r0 + CH, REM)], msga_v.at[pl.ds(0, REM)])
    pltpu.sync_copy(msga_v.at[pl.ds(0, REM)],
                    acca_out.at[pl.ds(c * NPAD + r0 + CH, REM)])
    pltpu.sync_copy(accb_sh.at[pl.ds(r0, CH)], msgb_v)
    pltpu.sync_copy(msgb_v, accb_out.at[pl.ds(c * NPAD + r0, CH)])
    pltpu.sync_copy(accb_sh.at[pl.ds(r0 + CH, REM)], msgb_v.at[pl.ds(0, REM)])
    pltpu.sync_copy(msgb_v.at[pl.ds(0, REM)],
                    accb_out.at[pl.ds(c * NPAD + r0 + CH, REM)])


def _run_edge_agg2(ridx2, dst, t2a, t2b):
    zcnt = jnp.zeros((CH,), jnp.float32)
    mesh = plsc.VectorSubcoreMesh(core_axis_name="c", subcore_axis_name="s")
    f = functools.partial(
        pl.kernel,
        out_type=[
            jax.ShapeDtypeStruct((NUM_CORES * NPAD,), jnp.float32),
            jax.ShapeDtypeStruct((NUM_CORES * NPAD,), jnp.float32),
        ],
        mesh=mesh,
        scratch_types=[
            pltpu.VMEM((CH,), jnp.int32),
            pltpu.VMEM((CH,), jnp.int32),
            pltpu.VMEM((CH,), jnp.float32),
            pltpu.VMEM((CH,), jnp.float32),
            pltpu.VMEM((CH,), jnp.int32),
            pltpu.VMEM((CH,), jnp.int32),
            pltpu.VMEM((CH,), jnp.float32),
            pltpu.VMEM((CH,), jnp.float32),
            pltpu.VMEM_SHARED((NPAD,), jnp.float32),
            pltpu.VMEM_SHARED((NPAD,), jnp.float32),
            pltpu.SemaphoreType.DMA,
            pltpu.SemaphoreType.DMA,
            pltpu.SemaphoreType.DMA,
            pltpu.SemaphoreType.DMA,
            pltpu.SemaphoreType.DMA,
            pltpu.SemaphoreType.DMA,
        ],
        compiler_params=_SC_PARAMS,
    )(_edge_agg2)
    return f(ridx2, dst, t2a, t2b, zcnt)


def _fin1_body(sums_ref, cnt_ref, rootT_ref, biasT_ref, a2x_ref, b2y_ref,
               xt_ref, w2t_ref):
    sums = sums_ref[0] + sums_ref[1]
    cinv = 1.0 / jnp.maximum(cnt_ref[0] + cnt_ref[1], 1.0)
    xt = jax.nn.relu(sums.T * cinv + rootT_ref[...] + biasT_ref[...])
    xt_ref[...] = xt
    w2t_ref[...] = jnp.dot(a2x_ref[...], b2y_ref[...],
                           preferred_element_type=jnp.float32)


def _finalize1(sums_p, cnt_p, root1t, bias1t, att2x, basis2y):
    grid = NPAD // BLKC
    return pl.pallas_call(
        _fin1_body,
        grid=(grid,),
        in_specs=[
            pl.BlockSpec((NUM_CORES, BLKC, H), lambda i: (0, i, 0)),
            pl.BlockSpec((NUM_CORES, 1, BLKC), lambda i: (0, 0, i)),
            pl.BlockSpec((H, BLKC), lambda i: (0, i)),
            pl.BlockSpec((H, 1), lambda i: (0, 0)),
            pl.BlockSpec((R * C, B * C), lambda i: (0, 0)),
            pl.BlockSpec((B * C, H), lambda i: (0, 0)),
        ],
        out_specs=[
            pl.BlockSpec((H, BLKC), lambda i: (0, i)),
            pl.BlockSpec((R * C, H), lambda i: (0, 0)),
        ],
        out_shape=[
            jax.ShapeDtypeStruct((H, NPAD), jnp.float32),
            jax.ShapeDtypeStruct((R * C, H), jnp.float32),
        ],
    )(sums_p, cnt_p.reshape(NUM_CORES, 1, NPAD), root1t, bias1t, att2x,
      basis2y)


_BLK2 = NPAD // 2


def _t2_body(xt_ref, w2t3_ref, a_ref, b_ref):
    xt = xt_ref[...]
    a_ref[...] = jnp.dot(w2t3_ref[0, 0:1, :], xt,
                         preferred_element_type=jnp.float32).reshape(-1)
    b_ref[...] = jnp.dot(w2t3_ref[0, 1:2, :], xt,
                         preferred_element_type=jnp.float32).reshape(-1)


def _build_t2(xt, w2t):
    return pl.pallas_call(
        _t2_body,
        grid=(2, R),
        in_specs=[
            pl.BlockSpec((H, _BLK2), lambda i, t: (0, i)),
            pl.BlockSpec((1, C, H), lambda i, t: (t, 0, 0)),
        ],
        out_specs=[
            pl.BlockSpec((_BLK2,), lambda i, t: (t * 2 + i,)),
            pl.BlockSpec((_BLK2,), lambda i, t: (t * 2 + i,)),
        ],
        out_shape=[
            jax.ShapeDtypeStruct((R * NPAD,), jnp.float32),
            jax.ShapeDtypeStruct((R * NPAD,), jnp.float32),
        ],
    )(xt, w2t.reshape(R, C, H))


def _fin2_body(acca_ref, accb_ref, cnt_ref, xt_ref, root2t_ref, bias2t_ref,
               out_ref):
    acc = jnp.concatenate(
        [acca_ref[0] + acca_ref[1], accb_ref[0] + accb_ref[1]], axis=0)
    cinv = 1.0 / jnp.maximum(cnt_ref[0] + cnt_ref[1], 1.0)
    z = acc * cinv + jnp.dot(root2t_ref[...], xt_ref[...],
                             preferred_element_type=jnp.float32)         + bias2t_ref[...]
    m = jnp.max(z, axis=0, keepdims=True)
    lse = m + jnp.log(jnp.sum(jnp.exp(z - m), axis=0, keepdims=True))
    out_ref[...] = z - lse


def _finalize2(acc2a_p, acc2b_p, cnt_p, xt, root2t, bias2t):
    grid = NPAD // BLKC
    return pl.pallas_call(
        _fin2_body,
        grid=(grid,),
        in_specs=[
            pl.BlockSpec((NUM_CORES, 1, BLKC), lambda i: (0, 0, i)),
            pl.BlockSpec((NUM_CORES, 1, BLKC), lambda i: (0, 0, i)),
            pl.BlockSpec((NUM_CORES, 1, BLKC), lambda i: (0, 0, i)),
            pl.BlockSpec((H, BLKC), lambda i: (0, i)),
            pl.BlockSpec((C, H), lambda i: (0, 0)),
            pl.BlockSpec((C, 1), lambda i: (0, 0)),
        ],
        out_specs=pl.BlockSpec((C, BLKC), lambda i: (0, i)),
        out_shape=jax.ShapeDtypeStruct((C, NPAD), jnp.float32),
    )(acc2a_p.reshape(NUM_CORES, 1, NPAD), acc2b_p.reshape(NUM_CORES, 1, NPAD),
      cnt_p.reshape(NUM_CORES, 1, NPAD), xt, root2t, bias2t)


def kernel(edge_index, edge_type, edge_norm, basis1, att1, root1, bias1,
           basis2, att2, root2, bias2):
    del edge_norm
    src = edge_index[0]
    dst = edge_index[1]

    w1flat = _build_w1(att1.reshape(R, 1, B), basis1.reshape(B, N * H))
    w_rows = w1flat.reshape(R * NROWS, H)

    src2d = src.reshape(12800, 125)
    typ2d = edge_type.reshape(12800, 125)
    ridx1_2d, ridx2_2d = _build_indices(src2d, typ2d)
    ridx1 = ridx1_2d.reshape(E)
    ridx2 = ridx2_2d.reshape(E)

    sums_p, cnt_p = _run_edge_agg1(ridx1, dst, w_rows)
    cnt_p = cnt_p.reshape(NUM_CORES, NPAD)

    att2x = (att2[:, None, :, None] *
             jnp.eye(C, dtype=jnp.float32)[None, :, None, :]).reshape(
                 R * C, B * C)
    basis2y = jnp.transpose(basis2, (0, 2, 1)).reshape(B * C, H)
    root1t = jnp.pad(root1.T, ((0, 0), (0, NPAD - N)))

    xt, w2t = _finalize1(sums_p, cnt_p, root1t, bias1.reshape(H, 1),
                         att2x, basis2y)
    t2a, t2b = _build_t2(xt, w2t)

    acc2a_p, acc2b_p = _run_edge_agg2(ridx2, dst, t2a, t2b)

    outt = _finalize2(acc2a_p, acc2b_p, cnt_p, xt, root2.T,
                      bias2.reshape(C, 1))
    return outt.T[:N]
